# Initial kernel scaffold; baseline (speedup 1.0000x reference)
import jax
import jax.numpy as jnp
from jax.experimental import pallas as pl


def kernel(Z, pair_diff, pair_i, pair_j, atom_batch, embed, pp_w1, pp_b1, pp_w2, pp_b2, pi_w1, pi_b1, pi_w2, pi_b2, ii_w1, ii_b1, ii_w2, ii_b2, ro_w1, ro_b1, ro_w2, ro_b2, ro_w3, ro_b3):
    raise NotImplementedError("write your pallas kernel here")


# speedup vs baseline: 1.4277x; 1.4277x over previous
import functools

import jax
import jax.numpy as jnp
from jax import lax
from jax.experimental import pallas as pl
from jax.experimental.pallas import tpu as pltpu
from jax.experimental.pallas import tpu_sc as plsc

_N_NODES = 10000
_N_EDGES = 160000
_N_BASIS = 10
_R_MAX = 5.0
_DEPTH = 3
_DIM = 64
_N_MOL = 128
_MAX_Z = 100

_NC = 2
_NS = 16
_NW = _NC * _NS
_EPW = _N_EDGES // _NW
_CH = 128
_NFULL = _EPW // _CH
_TAIL = _EPW - _NFULL * _CH

_BE = 2000


def _sc_mesh():
    return plsc.VectorSubcoreMesh(
        core_axis_name="c", subcore_axis_name="s", num_cores=_NC, num_subcores=_NS
    )


@functools.lru_cache(maxsize=None)
def _build_sc_gather():
    return functools.partial(
        pl.kernel,
        out_type=[
            jax.ShapeDtypeStruct((_N_EDGES, _DIM), jnp.float32),
            jax.ShapeDtypeStruct((_N_EDGES, _DIM), jnp.float32),
        ],
        mesh=_sc_mesh(),
        compiler_params=pltpu.CompilerParams(use_tc_tiling_on_sc=False),
        scratch_types=[
            pltpu.VMEM((_CH,), jnp.int32),
            pltpu.VMEM((_CH, _DIM), jnp.float32),
            pltpu.VMEM((_TAIL,), jnp.int32),
            pltpu.VMEM((_TAIL, _DIM), jnp.float32),
            pltpu.SemaphoreType.DMA,
        ],
    )(_sc_gather_body)


def _sc_gather(pp, pi, pj):
    return _build_sc_gather()(pp, pi, pj)


def _sc_gather_body(pp, pi, pj, gi, gj, idx_v, rows_v, idx_t, rows_t, sem):
    c = lax.axis_index("c")
    s = lax.axis_index("s")
    wid = s * _NC + c
    base0 = wid * _EPW

    def body(ci, carry):
        base = pl.multiple_of(base0 + ci * _CH, 8)
        pltpu.sync_copy(pi.at[pl.ds(base, _CH)], idx_v)
        pltpu.async_copy(pp.at[idx_v], rows_v, sem).wait()
        pltpu.sync_copy(rows_v, gi.at[pl.ds(base, _CH)])
        pltpu.sync_copy(pj.at[pl.ds(base, _CH)], idx_v)
        pltpu.async_copy(pp.at[idx_v], rows_v, sem).wait()
        pltpu.sync_copy(rows_v, gj.at[pl.ds(base, _CH)])
        return carry

    lax.fori_loop(0, _NFULL, body, 0)

    baset = pl.multiple_of(base0 + _NFULL * _CH, 8)
    pltpu.sync_copy(pi.at[pl.ds(baset, _TAIL)], idx_t)
    pltpu.async_copy(pp.at[idx_t], rows_t, sem).wait()
    pltpu.sync_copy(rows_t, gi.at[pl.ds(baset, _TAIL)])
    pltpu.sync_copy(pj.at[pl.ds(baset, _TAIL)], idx_t)
    pltpu.async_copy(pp.at[idx_t], rows_t, sem).wait()
    pltpu.sync_copy(rows_t, gj.at[pl.ds(baset, _TAIL)])


@functools.lru_cache(maxsize=None)
def _build_sc_scatter():
    return functools.partial(
        pl.kernel,
        out_type=jax.ShapeDtypeStruct((_NC, _N_NODES, _DIM), jnp.float32),
        mesh=_sc_mesh(),
        compiler_params=pltpu.CompilerParams(use_tc_tiling_on_sc=False),
        scratch_types=[
            pltpu.VMEM_SHARED((_N_NODES, _DIM), jnp.float32),
            pltpu.VMEM((_CH,), jnp.int32),
            pltpu.VMEM((_CH, _DIM), jnp.float32),
            pltpu.VMEM((_TAIL,), jnp.int32),
            pltpu.VMEM((_TAIL, _DIM), jnp.float32),
        ],
    )(_sc_scatter_body)


def _sc_scatter(zeros_hbm, pi, inter):
    return _build_sc_scatter()(zeros_hbm, pi, inter)


def _sc_scatter_body(zeros_hbm, pi, inter, out, shared, idx_v, rows_v, idx_t, rows_t):
    c = lax.axis_index("c")
    s = lax.axis_index("s")

    @pl.when(s == 0)
    def _zero():
        pltpu.sync_copy(zeros_hbm, shared)

    plsc.subcore_barrier()

    wid = s * _NC + c
    base0 = wid * _EPW

    def body(ci, carry):
        base = pl.multiple_of(base0 + ci * _CH, 8)
        pltpu.sync_copy(pi.at[pl.ds(base, _CH)], idx_v)
        pltpu.sync_copy(inter.at[pl.ds(base, _CH)], rows_v)
        pltpu.sync_copy(rows_v, shared.at[idx_v], add=True)
        return carry

    lax.fori_loop(0, _NFULL, body, 0)

    baset = pl.multiple_of(base0 + _NFULL * _CH, 8)
    pltpu.sync_copy(pi.at[pl.ds(baset, _TAIL)], idx_t)
    pltpu.sync_copy(inter.at[pl.ds(baset, _TAIL)], rows_t)
    pltpu.sync_copy(rows_t, shared.at[idx_t], add=True)

    plsc.subcore_barrier()

    @pl.when(s == 0)
    def _writeback():
        pltpu.sync_copy(shared, out.at[c])


def _node0(Z2, embed, w1, b1, w2, b2):
    def body(z_ref, e_ref, w1_ref, b1_ref, w2_ref, b2_ref, p_ref, pp_ref):
        z = z_ref[...]
        iota = lax.broadcasted_iota(jnp.int32, (_N_NODES, _MAX_Z), 1)
        oh = (z == iota).astype(jnp.float32)
        p = jnp.dot(oh, e_ref[...], preferred_element_type=jnp.float32)
        h = jnp.tanh(jnp.dot(p, w1_ref[...], preferred_element_type=jnp.float32)
                     + b1_ref[...])
        h = jnp.tanh(jnp.dot(h, w2_ref[...], preferred_element_type=jnp.float32)
                     + b2_ref[...])
        p_ref[...] = p
        pp_ref[...] = h

    return pl.pallas_call(
        body,
        out_shape=[
            jax.ShapeDtypeStruct((_N_NODES, _DIM), jnp.float32),
            jax.ShapeDtypeStruct((_N_NODES, _DIM), jnp.float32),
        ],
    )(Z2, embed, w1, b1, w2, b2)


def _node(p, parts, w1, b1, w2, b2):
    def body(p_ref, parts_ref, w1_ref, b1_ref, w2_ref, b2_ref, p_out, pp_ref):
        pnew = p_ref[...] + parts_ref[0] + parts_ref[1]
        h = jnp.tanh(jnp.dot(pnew, w1_ref[...], preferred_element_type=jnp.float32)
                     + b1_ref[...])
        h = jnp.tanh(jnp.dot(h, w2_ref[...], preferred_element_type=jnp.float32)
                     + b2_ref[...])
        p_out[...] = pnew
        pp_ref[...] = h

    return pl.pallas_call(
        body,
        out_shape=[
            jax.ShapeDtypeStruct((_N_NODES, _DIM), jnp.float32),
            jax.ShapeDtypeStruct((_N_NODES, _DIM), jnp.float32),
        ],
    )(p, parts, w1, b1, w2, b2)


def _edge(pair_diff, gi, gj, w1a, w1b, b1, w2, b2, iw1, ib1, iw2, ib2):
    sigma = _R_MAX / _N_BASIS

    def body(pd_ref, gi_ref, gj_ref, w1a_ref, w1b_ref, b1_ref, w2_ref, b2_ref,
             iw1_ref, ib1_ref, iw2_ref, ib2_ref, out_ref):
        pd = pd_ref[...]
        r2 = jnp.sum(pd * pd, axis=1, keepdims=True)
        r = jnp.sqrt(r2 + 1e-12)
        fc = 0.5 * (jnp.cos(jnp.pi * r / _R_MAX) + 1.0)
        fc = jnp.where(r < _R_MAX, fc, 0.0)
        h1 = jnp.tanh(
            jnp.dot(gi_ref[...], w1a_ref[...], preferred_element_type=jnp.float32)
            + jnp.dot(gj_ref[...], w1b_ref[...], preferred_element_type=jnp.float32)
            + b1_ref[...])
        h2 = jnp.dot(h1, w2_ref[...], preferred_element_type=jnp.float32) + b2_ref[...]
        acc = jnp.zeros((_BE, _DIM), jnp.float32)
        for b in range(_N_BASIS):
            mu = _R_MAX * b / (_N_BASIS - 1)
            wb = jnp.exp(-0.5 * ((r - mu) / sigma) ** 2) * fc
            acc = acc + h2[:, b * _DIM:(b + 1) * _DIM] * wb
        t = jnp.tanh(jnp.dot(acc, iw1_ref[...], preferred_element_type=jnp.float32)
                     + ib1_ref[...])
        t = jnp.tanh(jnp.dot(t, iw2_ref[...], preferred_element_type=jnp.float32)
                     + ib2_ref[...])
        out_ref[...] = t * fc

    n_blocks = _N_EDGES // _BE
    const = lambda shape: pl.BlockSpec(shape, lambda i: (0, 0))
    return pl.pallas_call(
        body,
        grid=(n_blocks,),
        in_specs=[
            pl.BlockSpec((_BE, 3), lambda i: (i, 0)),
            pl.BlockSpec((_BE, _DIM), lambda i: (i, 0)),
            pl.BlockSpec((_BE, _DIM), lambda i: (i, 0)),
            const((_DIM, _DIM)),
            const((_DIM, _DIM)),
            const((1, _DIM)),
            const((_DIM, _N_BASIS * _DIM)),
            const((1, _N_BASIS * _DIM)),
            const((_DIM, _DIM)),
            const((1, _DIM)),
            const((_DIM, _DIM)),
            const((1, _DIM)),
        ],
        out_specs=pl.BlockSpec((_BE, _DIM), lambda i: (i, 0)),
        out_shape=jax.ShapeDtypeStruct((_N_EDGES, _DIM), jnp.float32),
    )(pair_diff, gi, gj, w1a, w1b, b1, w2, b2, iw1, ib1, iw2, ib2)


def _readout(p, parts, ab2, w1, b1, w2, b2, w3, b3):
    def body(p_ref, parts_ref, ab_ref, w1_ref, b1_ref, w2_ref, b2_ref,
             w3_ref, b3_ref, out_ref):
        pnew = p_ref[...] + parts_ref[0] + parts_ref[1]
        x = jnp.tanh(jnp.dot(pnew, w1_ref[...], preferred_element_type=jnp.float32)
                     + b1_ref[...])
        x = jnp.tanh(jnp.dot(x, w2_ref[...], preferred_element_type=jnp.float32)
                     + b2_ref[...])
        y = jnp.dot(x, w3_ref[...], preferred_element_type=jnp.float32) + b3_ref[...]
        ab = ab_ref[...]
        iota = lax.broadcasted_iota(jnp.int32, (_N_NODES, _N_MOL), 1)
        contrib = jnp.where(ab == iota, y, 0.0)
        out_ref[...] = jnp.sum(contrib, axis=0, keepdims=True)

    return pl.pallas_call(
        body,
        out_shape=jax.ShapeDtypeStruct((1, _N_MOL), jnp.float32),
    )(p, parts, ab2, w1, b1, w2, b2, w3, b3)


def kernel(Z, pair_diff, pair_i, pair_j, atom_batch, embed, pp_w1, pp_b1,
           pp_w2, pp_b2, pi_w1, pi_b1, pi_w2, pi_b2, ii_w1, ii_b1, ii_w2,
           ii_b2, ro_w1, ro_b1, ro_w2, ro_b2, ro_w3, ro_b3):
    Z2 = Z.reshape(_N_NODES, 1).astype(jnp.int32)
    ab2 = atom_batch.reshape(_N_NODES, 1).astype(jnp.int32)
    pair_i = pair_i.astype(jnp.int32)
    pair_j = pair_j.astype(jnp.int32)
    zeros = jnp.zeros((_N_NODES, _DIM), jnp.float32)

    p = None
    parts = None
    for d in range(_DEPTH):
        if d == 0:
            p, pp = _node0(Z2, embed, pp_w1[0], pp_b1[0].reshape(1, _DIM),
                           pp_w2[0], pp_b2[0].reshape(1, _DIM))
        else:
            p, pp = _node(p, parts, pp_w1[d], pp_b1[d].reshape(1, _DIM),
                          pp_w2[d], pp_b2[d].reshape(1, _DIM))
        gi, gj = _sc_gather(pp, pair_i, pair_j)
        inter = _edge(
            pair_diff, gi, gj,
            pi_w1[d, :_DIM], pi_w1[d, _DIM:], pi_b1[d].reshape(1, _DIM),
            pi_w2[d], pi_b2[d].reshape(1, _N_BASIS * _DIM),
            ii_w1[d], ii_b1[d].reshape(1, _DIM),
            ii_w2[d], ii_b2[d].reshape(1, _DIM),
        )
        parts = _sc_scatter(zeros, pair_i, inter)

    out = _readout(p, parts, ab2,
                   ro_w1, ro_b1.reshape(1, _DIM),
                   ro_w2, ro_b2.reshape(1, 32),
                   ro_w3, ro_b3.reshape(1, 1))
    return out[0]
